# baseline (speedup 1.0000x reference)
import jax
import jax.numpy as jnp
from jax import lax
from jax.experimental import pallas as pl
from jax.experimental.pallas import tpu as pltpu
from jax.experimental.pallas import tpu_sc as plsc

VOCAB = 30522
HIDDEN = 1024
BATCH = 4
SEQ = 2048
EPS = 1e-12

TOK = BATCH * SEQ
_INFO = plsc.get_sparse_core_info()
NC = _INFO.num_cores
NS = _INFO.num_subcores
NW = NC * NS
PER_W = TOK // NW
CH = 32
NCH = PER_W // CH


def _sc_gather_body(ids_hbm, table_hbm, out_hbm, idx_v, rows_v, gsem, ssem):
    wid = lax.axis_index("s") * NC + lax.axis_index("c")
    base = wid * PER_W
    pltpu.sync_copy(ids_hbm.at[pl.ds(base, PER_W)], idx_v)
    g = [None] * NCH
    s = [None] * NCH
    for k in range(NCH):
        if k >= 2:
            s[k - 2].wait()
        g[k] = pltpu.async_copy(
            table_hbm.at[idx_v.at[pl.ds(k * CH, CH)]], rows_v.at[k % 2], gsem)
        if k >= 1:
            g[k - 1].wait()
            s[k - 1] = pltpu.async_copy(
                rows_v.at[(k - 1) % 2],
                out_hbm.at[pl.ds(base + (k - 1) * CH, CH)], ssem)
    g[NCH - 1].wait()
    s[NCH - 1] = pltpu.async_copy(
        rows_v.at[(NCH - 1) % 2],
        out_hbm.at[pl.ds(base + (NCH - 1) * CH, CH)], ssem)
    s[NCH - 2].wait()
    s[NCH - 1].wait()


_sc_gather = pl.kernel(
    _sc_gather_body,
    mesh=plsc.VectorSubcoreMesh(core_axis_name="c", subcore_axis_name="s"),
    out_type=jax.ShapeDtypeStruct((TOK, HIDDEN), jnp.float32),
    scratch_types=[
        pltpu.VMEM((PER_W,), jnp.int32),
        pltpu.VMEM((2, CH, HIDDEN), jnp.float32),
        pltpu.SemaphoreType.DMA,
        pltpu.SemaphoreType.DMA,
    ],
)

BS = 2048
SB = SEQ // BS


def _tc_ln_body(g_ref, pos_ref, tt_ref, ttab_ref, gamma_ref, beta_ref, o_ref):
    x = g_ref[...] + pos_ref[...]
    ids = tt_ref[0, 0, :]
    w = jnp.clip(ids, 0, 1).astype(jnp.float32)[:, None]
    tt0 = ttab_ref[0, :][None, :]
    tt1 = ttab_ref[1, :][None, :]
    x = x + tt0 + w * (tt1 - tt0)
    mean = jnp.mean(x, axis=-1, keepdims=True)
    xc = x - mean
    var = jnp.mean(xc * xc, axis=-1, keepdims=True)
    y = xc * lax.rsqrt(var + EPS)
    o_ref[0] = y * gamma_ref[0, :][None, :] + beta_ref[0, :][None, :]


_tc_ln = pl.pallas_call(
    _tc_ln_body,
    grid=(SB, BATCH),
    in_specs=[
        pl.BlockSpec((BS, HIDDEN), lambda i, j: (j * SB + i, 0)),
        pl.BlockSpec((BS, HIDDEN), lambda i, j: (i, 0)),
        pl.BlockSpec((1, 1, BS), lambda i, j: (j * SB + i, 0, 0)),
        pl.BlockSpec((2, HIDDEN), lambda i, j: (0, 0)),
        pl.BlockSpec((1, HIDDEN), lambda i, j: (0, 0)),
        pl.BlockSpec((1, HIDDEN), lambda i, j: (0, 0)),
    ],
    out_specs=pl.BlockSpec((1, BS, HIDDEN), lambda i, j: (j, i, 0)),
    out_shape=jax.ShapeDtypeStruct((BATCH, SEQ, HIDDEN), jnp.float32),
)


@jax.jit
def kernel(input_ids, token_type_ids, word_embeddings, position_embeddings,
           token_type_embeddings, gamma, beta):
    ids = input_ids.reshape(-1).astype(jnp.int32)
    gathered = _sc_gather(ids, word_embeddings)
    tt = token_type_ids.reshape(TOK // BS, 1, BS).astype(jnp.int32)
    return _tc_ln(gathered, position_embeddings, tt, token_type_embeddings,
                  gamma.reshape(1, HIDDEN), beta.reshape(1, HIDDEN))


# speedup vs baseline: 1.0190x; 1.0190x over previous
import jax
import jax.numpy as jnp
from jax import lax
from jax.experimental import pallas as pl
from jax.experimental.pallas import tpu as pltpu
from jax.experimental.pallas import tpu_sc as plsc

VOCAB = 30522
HIDDEN = 1024
BATCH = 4
SEQ = 2048
EPS = 1e-12

TOK = BATCH * SEQ
_INFO = plsc.get_sparse_core_info()
NC = _INFO.num_cores
NS = _INFO.num_subcores
NW = NC * NS
PER_W = TOK // NW
CH = 32
NCH = PER_W // CH


def _sc_gather_body(ids_hbm, table_hbm, out_hbm, idx_v, rows_v, gsem, ssem):
    wid = lax.axis_index("s") * NC + lax.axis_index("c")
    base = wid * PER_W
    pltpu.sync_copy(ids_hbm.at[pl.ds(base, PER_W)], idx_v)
    g = [None] * NCH
    s = [None] * NCH
    for k in range(NCH):
        if k >= 2:
            s[k - 2].wait()
        g[k] = pltpu.async_copy(
            table_hbm.at[idx_v.at[pl.ds(k * CH, CH)]], rows_v.at[k % 2], gsem)
        if k >= 1:
            g[k - 1].wait()
            s[k - 1] = pltpu.async_copy(
                rows_v.at[(k - 1) % 2],
                out_hbm.at[pl.ds(base + (k - 1) * CH, CH)], ssem)
    g[NCH - 1].wait()
    s[NCH - 1] = pltpu.async_copy(
        rows_v.at[(NCH - 1) % 2],
        out_hbm.at[pl.ds(base + (NCH - 1) * CH, CH)], ssem)
    s[NCH - 2].wait()
    s[NCH - 1].wait()


_sc_gather = pl.kernel(
    _sc_gather_body,
    mesh=plsc.VectorSubcoreMesh(core_axis_name="c", subcore_axis_name="s"),
    out_type=jax.ShapeDtypeStruct((TOK, HIDDEN), jnp.float32),
    scratch_types=[
        pltpu.VMEM((PER_W,), jnp.int32),
        pltpu.VMEM((2, CH, HIDDEN), jnp.float32),
        pltpu.SemaphoreType.DMA,
        pltpu.SemaphoreType.DMA,
    ],
)

BS = 1024
SB = SEQ // BS


def _tc_ln_body(g_ref, pos_ref, tt_ref, ttab_ref, gamma_ref, beta_ref, o_ref):
    x = g_ref[...] + pos_ref[...]
    ids = tt_ref[0, 0, :]
    w = jnp.clip(ids, 0, 1).astype(jnp.float32)[:, None]
    tt0 = ttab_ref[0, :][None, :]
    tt1 = ttab_ref[1, :][None, :]
    x = x + tt0 + w * (tt1 - tt0)
    mean = jnp.mean(x, axis=-1, keepdims=True)
    xc = x - mean
    var = jnp.mean(xc * xc, axis=-1, keepdims=True)
    y = xc * lax.rsqrt(var + EPS)
    o_ref[0] = y * gamma_ref[0, :][None, :] + beta_ref[0, :][None, :]


_tc_ln = pl.pallas_call(
    _tc_ln_body,
    grid=(SB, BATCH),
    in_specs=[
        pl.BlockSpec((BS, HIDDEN), lambda i, j: (j * SB + i, 0)),
        pl.BlockSpec((BS, HIDDEN), lambda i, j: (i, 0)),
        pl.BlockSpec((1, 1, BS), lambda i, j: (j * SB + i, 0, 0)),
        pl.BlockSpec((2, HIDDEN), lambda i, j: (0, 0)),
        pl.BlockSpec((1, HIDDEN), lambda i, j: (0, 0)),
        pl.BlockSpec((1, HIDDEN), lambda i, j: (0, 0)),
    ],
    out_specs=pl.BlockSpec((1, BS, HIDDEN), lambda i, j: (j, i, 0)),
    out_shape=jax.ShapeDtypeStruct((BATCH, SEQ, HIDDEN), jnp.float32),
)


@jax.jit
def kernel(input_ids, token_type_ids, word_embeddings, position_embeddings,
           token_type_embeddings, gamma, beta):
    ids = input_ids.reshape(-1).astype(jnp.int32)
    gathered = _sc_gather(ids, word_embeddings)
    tt = token_type_ids.reshape(TOK // BS, 1, BS).astype(jnp.int32)
    return _tc_ln(gathered, position_embeddings, tt, token_type_embeddings,
                  gamma.reshape(1, HIDDEN), beta.reshape(1, HIDDEN))


# speedup vs baseline: 1.3805x; 1.3548x over previous
import jax
import jax.numpy as jnp
from jax import lax
from jax.experimental import pallas as pl
from jax.experimental.pallas import tpu as pltpu
from jax.experimental.pallas import tpu_sc as plsc

VOCAB = 30522
HIDDEN = 1024
BATCH = 4
SEQ = 2048
EPS = 1e-12

TOK = BATCH * SEQ
_INFO = plsc.get_sparse_core_info()
NC = _INFO.num_cores
NS = _INFO.num_subcores
NW = NC * NS
PER_W = TOK // NW
CH = 32
NCH = PER_W // CH


NBUF = 3


def _sc_gather_body(ids_hbm, table_hbm, out_hbm, idx_v, rows_v, gsem, ssem):
    wid = lax.axis_index("s") * NC + lax.axis_index("c")
    base = wid * PER_W
    pltpu.sync_copy(ids_hbm.at[pl.ds(base, PER_W)], idx_v)
    g = [None] * NCH
    s = [None] * NCH

    def start_gather(k):
        return pltpu.async_copy(
            table_hbm.at[idx_v.at[pl.ds(k * CH, CH)]],
            rows_v.at[k % NBUF], gsem)

    g[0] = start_gather(0)
    g[1] = start_gather(1)
    for k in range(NCH):
        if k + 2 < NCH:
            if k - 1 >= 0:
                s[k - 1].wait()
            g[k + 2] = start_gather(k + 2)
        g[k].wait()
        s[k] = pltpu.async_copy(
            rows_v.at[k % NBUF],
            out_hbm.at[pl.ds(base + k * CH, CH)], ssem)
    s[NCH - 2].wait()
    s[NCH - 1].wait()


_sc_gather = pl.kernel(
    _sc_gather_body,
    mesh=plsc.VectorSubcoreMesh(core_axis_name="c", subcore_axis_name="s"),
    out_type=jax.ShapeDtypeStruct((TOK, HIDDEN), jnp.float32),
    scratch_types=[
        pltpu.VMEM((PER_W,), jnp.int32),
        pltpu.VMEM((NBUF, CH, HIDDEN), jnp.float32),
        pltpu.SemaphoreType.DMA,
        pltpu.SemaphoreType.DMA,
    ],
)

BS = 1024
SB = SEQ // BS


def _tc_ln_body(g_ref, pos_ref, tt_ref, ttab_ref, gamma_ref, beta_ref, o_ref):
    x = g_ref[...] + pos_ref[...]
    ids = tt_ref[0, 0, :]
    w = jnp.clip(ids, 0, 1).astype(jnp.float32)[:, None]
    tt0 = ttab_ref[0, :][None, :]
    tt1 = ttab_ref[1, :][None, :]
    x = x + tt0 + w * (tt1 - tt0)
    mean = jnp.mean(x, axis=-1, keepdims=True)
    xc = x - mean
    var = jnp.mean(xc * xc, axis=-1, keepdims=True)
    y = xc * lax.rsqrt(var + EPS)
    o_ref[0] = y * gamma_ref[0, :][None, :] + beta_ref[0, :][None, :]


_tc_ln = pl.pallas_call(
    _tc_ln_body,
    grid=(SB, BATCH),
    in_specs=[
        pl.BlockSpec((BS, HIDDEN), lambda i, j: (j * SB + i, 0)),
        pl.BlockSpec((BS, HIDDEN), lambda i, j: (i, 0)),
        pl.BlockSpec((1, 1, BS), lambda i, j: (j * SB + i, 0, 0)),
        pl.BlockSpec((2, HIDDEN), lambda i, j: (0, 0)),
        pl.BlockSpec((1, HIDDEN), lambda i, j: (0, 0)),
        pl.BlockSpec((1, HIDDEN), lambda i, j: (0, 0)),
    ],
    out_specs=pl.BlockSpec((1, BS, HIDDEN), lambda i, j: (j, i, 0)),
    out_shape=jax.ShapeDtypeStruct((BATCH, SEQ, HIDDEN), jnp.float32),
)


@jax.jit
def kernel(input_ids, token_type_ids, word_embeddings, position_embeddings,
           token_type_embeddings, gamma, beta):
    ids = input_ids.reshape(-1).astype(jnp.int32)
    gathered = _sc_gather(ids, word_embeddings)
    tt = token_type_ids.reshape(TOK // BS, 1, BS).astype(jnp.int32)
    return _tc_ln(gathered, position_embeddings, tt, token_type_embeddings,
                  gamma.reshape(1, HIDDEN), beta.reshape(1, HIDDEN))
